# baseline (speedup 1.0000x reference)
import functools

import jax
import jax.numpy as jnp
from jax import lax
from jax.experimental import pallas as pl
from jax.experimental.pallas import tpu as pltpu
from jax.experimental.pallas import tpu_sc as plsc

_N_FIELDS = 26
_VOCAB = 100000
_EMB_DIM = 32
_BATCH = 16384
_LANES = 16

_N_CHUNKS = 8
_RING = 2


@functools.cache
def _build_sc_kernel():
    info = plsc.get_sparse_core_info()
    nc, ns = info.num_cores, info.num_subcores
    nw = nc * ns
    rpw = _BATCH * _N_FIELDS // nw
    ch = rpw // _N_CHUNKS
    period = 13
    vecs_per_chunk = ch // _LANES

    mesh = plsc.VectorSubcoreMesh(core_axis_name="c", subcore_axis_name="s")

    @functools.partial(
        pl.kernel,
        mesh=mesh,
        out_type=jax.ShapeDtypeStruct((_BATCH * _N_FIELDS, _EMB_DIM), jnp.float32),
        scratch_types=[
            pltpu.VMEM((_N_CHUNKS, ch), jnp.int32),
            pltpu.VMEM((period * _LANES,), jnp.int32),
            pltpu.VMEM((_RING, ch, _EMB_DIM), jnp.float32),
            pltpu.SemaphoreType.DMA,
            pltpu.SemaphoreType.DMA,
        ],
        compiler_params=pltpu.CompilerParams(use_tc_tiling_on_sc=False),
    )
    def sc_embed(idx_hbm, tab_hbm, out_hbm, idx_v, offs_v, rows_v, gsem, wsem):
        wid = lax.axis_index("s") * nc + lax.axis_index("c")
        p0 = wid * rpw

        pltpu.sync_copy(idx_hbm.at[wid], idx_v)

        def build_offs(k, c):
            lane = lax.iota(jnp.int32, _LANES) + k * _LANES
            offs_v[pl.ds(k * _LANES, _LANES)] = (lane % _N_FIELDS) * _VOCAB
            return c

        lax.fori_loop(0, period, build_offs, 0)

        def add_offs_chunk(c):
            def body(m, carry):
                for t in range(period):
                    sl = pl.ds((m * period + t) * _LANES, _LANES)
                    idx_v[c, sl] = idx_v[c, sl] + offs_v[pl.ds(t * _LANES, _LANES)]
                return carry

            lax.fori_loop(0, vecs_per_chunk // period, body, 0)

        def fire_gather(c):
            return pltpu.async_copy(tab_hbm.at[idx_v.at[c]], rows_v.at[c % _RING], gsem)

        gd = [None] * _N_CHUNKS
        wd = [None] * _N_CHUNKS
        for c in range(_RING):
            add_offs_chunk(c)
            gd[c] = fire_gather(c)
        for c in range(_N_CHUNKS):
            gd[c].wait()
            wd[c] = pltpu.async_copy(
                rows_v.at[c % _RING], out_hbm.at[pl.ds(p0 + c * ch, ch)], wsem
            )
            n = c + _RING
            if n < _N_CHUNKS:
                add_offs_chunk(n)
                wd[c].wait()
                gd[n] = fire_gather(n)
        for c in range(_N_CHUNKS - _RING, _N_CHUNKS):
            wd[c].wait()

    return sc_embed


def kernel(sparse_inputs, tables):
    nw = 32
    rpw = _BATCH * _N_FIELDS // nw
    ch = rpw // _N_CHUNKS
    idx3 = sparse_inputs.astype(jnp.int32).reshape(nw, _N_CHUNKS, ch)
    tab_flat = tables.reshape(_N_FIELDS * _VOCAB, _EMB_DIM)
    out = _build_sc_kernel()(idx3, tab_flat)
    return out.reshape(_BATCH, _N_FIELDS * _EMB_DIM)
